# baseline (speedup 1.0000x reference)
import jax
import jax.numpy as jnp
from jax.experimental import pallas as pl
from jax.experimental.pallas import tpu as pltpu

_CHUNK = 2000
_NBUF = 4


def _make_body(nchunk):
    def body(xh, wv, oh, xbuf, obuf, insem, outsem):
        for k in range(_NBUF):
            pltpu.make_async_copy(
                xh.at[pl.ds(k * _CHUNK, _CHUNK)], xbuf.at[k], insem.at[k]
            ).start()

        def step(i, carry):
            slot = jax.lax.rem(i, _NBUF)
            pltpu.make_async_copy(
                xh.at[pl.ds(i * _CHUNK, _CHUNK)], xbuf.at[slot], insem.at[slot]
            ).wait()
            r = jnp.dot(
                xbuf[slot], wv[...],
                precision=jax.lax.Precision.DEFAULT,
                preferred_element_type=jnp.float32,
            )

            @pl.when(i >= _NBUF)
            def _():
                pltpu.make_async_copy(
                    obuf.at[slot],
                    oh.at[pl.ds((i - _NBUF) * _CHUNK, _CHUNK)],
                    outsem.at[slot],
                ).wait()

            obuf[slot] = r.astype(jnp.bfloat16)
            pltpu.make_async_copy(
                obuf.at[slot], oh.at[pl.ds(i * _CHUNK, _CHUNK)], outsem.at[slot]
            ).start()

            @pl.when(i + _NBUF < nchunk)
            def _():
                pltpu.make_async_copy(
                    xh.at[pl.ds((i + _NBUF) * _CHUNK, _CHUNK)],
                    xbuf.at[slot],
                    insem.at[slot],
                ).start()

            return carry

        jax.lax.fori_loop(0, nchunk, step, 0)
        for i in range(max(nchunk - _NBUF, 0), nchunk):
            slot = i % _NBUF
            pltpu.make_async_copy(
                obuf.at[slot], oh.at[pl.ds(i * _CHUNK, _CHUNK)], outsem.at[slot]
            ).wait()

    return body


def kernel(x, W_cls, b_cls, W_box, b_box):
    if x.ndim > 2:
        x = x.reshape(x.shape[0], -1)
    N, D = x.shape
    C = W_cls.shape[0]
    B = W_box.shape[0]

    W = jnp.concatenate([W_cls, W_box], axis=0)
    Wp = jnp.zeros((128, D), x.dtype).at[: C + B].set(W).T

    pad = (-N) % _CHUNK
    if pad:
        x = jnp.pad(x, ((0, pad), (0, 0)))
    Np = N + pad
    nchunk = Np // _CHUNK

    out = pl.pallas_call(
        _make_body(nchunk),
        in_specs=[
            pl.BlockSpec(memory_space=pl.ANY),
            pl.BlockSpec(memory_space=pltpu.VMEM),
        ],
        out_specs=pl.BlockSpec(memory_space=pl.ANY),
        out_shape=jax.ShapeDtypeStruct((Np, 128), jnp.bfloat16),
        scratch_shapes=[
            pltpu.VMEM((_NBUF, _CHUNK, D), jnp.float32),
            pltpu.VMEM((_NBUF, _CHUNK, 128), jnp.bfloat16),
            pltpu.SemaphoreType.DMA((_NBUF,)),
            pltpu.SemaphoreType.DMA((_NBUF,)),
        ],
    )(x, Wp)

    sel_s = jnp.zeros((128, C), jnp.bfloat16).at[jnp.arange(C), jnp.arange(C)].set(1.0)
    sel_d = jnp.zeros((128, B), jnp.bfloat16).at[C + jnp.arange(B), jnp.arange(B)].set(1.0)
    scores = jnp.dot(out[:N], sel_s, preferred_element_type=jnp.float32) + b_cls
    deltas = jnp.dot(out[:N], sel_d, preferred_element_type=jnp.float32) + b_box
    return scores, deltas


# speedup vs baseline: 1.0137x; 1.0137x over previous
import jax
import jax.numpy as jnp
from jax.experimental import pallas as pl
from jax.experimental.pallas import tpu as pltpu

_CHUNK = 1000
_NBUF = 6


def _make_body(nchunk):
    def body(xh, wv, oh, xbuf, obuf, insem, outsem):
        for k in range(_NBUF):
            pltpu.make_async_copy(
                xh.at[pl.ds(k * _CHUNK, _CHUNK)], xbuf.at[k], insem.at[k]
            ).start()

        def step(i, carry):
            slot = jax.lax.rem(i, _NBUF)
            pltpu.make_async_copy(
                xh.at[pl.ds(i * _CHUNK, _CHUNK)], xbuf.at[slot], insem.at[slot]
            ).wait()
            r = jnp.dot(
                xbuf[slot], wv[...],
                precision=jax.lax.Precision.DEFAULT,
                preferred_element_type=jnp.float32,
            )

            obuf[i] = r.astype(jnp.bfloat16)
            pltpu.make_async_copy(
                obuf.at[i], oh.at[pl.ds(i * _CHUNK, _CHUNK)], outsem
            ).start()

            @pl.when(i + _NBUF < nchunk)
            def _():
                pltpu.make_async_copy(
                    xh.at[pl.ds((i + _NBUF) * _CHUNK, _CHUNK)],
                    xbuf.at[slot],
                    insem.at[slot],
                ).start()

            return carry

        jax.lax.fori_loop(0, nchunk, step, 0)
        for i in range(nchunk):
            pltpu.make_async_copy(
                obuf.at[i], oh.at[pl.ds(i * _CHUNK, _CHUNK)], outsem
            ).wait()

    return body


def kernel(x, W_cls, b_cls, W_box, b_box):
    if x.ndim > 2:
        x = x.reshape(x.shape[0], -1)
    N, D = x.shape
    C = W_cls.shape[0]
    B = W_box.shape[0]

    W = jnp.concatenate([W_cls, W_box], axis=0)
    Wp = jnp.zeros((128, D), x.dtype).at[: C + B].set(W).T

    pad = (-N) % _CHUNK
    if pad:
        x = jnp.pad(x, ((0, pad), (0, 0)))
    Np = N + pad
    nchunk = Np // _CHUNK

    out = pl.pallas_call(
        _make_body(nchunk),
        in_specs=[
            pl.BlockSpec(memory_space=pl.ANY),
            pl.BlockSpec(memory_space=pltpu.VMEM),
        ],
        out_specs=pl.BlockSpec(memory_space=pl.ANY),
        out_shape=jax.ShapeDtypeStruct((Np, 128), jnp.bfloat16),
        scratch_shapes=[
            pltpu.VMEM((_NBUF, _CHUNK, D), jnp.float32),
            pltpu.VMEM((Np // _CHUNK, _CHUNK, 128), jnp.bfloat16),
            pltpu.SemaphoreType.DMA((_NBUF,)),
            pltpu.SemaphoreType.DMA,
        ],
    )(x, Wp)

    sel_s = jnp.zeros((128, C), jnp.bfloat16).at[jnp.arange(C), jnp.arange(C)].set(1.0)
    sel_d = jnp.zeros((128, B), jnp.bfloat16).at[C + jnp.arange(B), jnp.arange(B)].set(1.0)
    scores = jnp.dot(out[:N], sel_s, preferred_element_type=jnp.float32) + b_cls
    deltas = jnp.dot(out[:N], sel_d, preferred_element_type=jnp.float32) + b_box
    return scores, deltas
